# baseline (speedup 1.0000x reference)
import functools

import jax
import jax.numpy as jnp
from jax import lax
from jax.experimental import pallas as pl
from jax.experimental.pallas import tpu as pltpu
from jax.experimental.pallas import tpu_sc as plsc

NC = 2
NS = 16
CH = 128


def _segsum_sc(n_pad, d2, k_tile, grp):
  rows_per_tile = n_pad // NS
  rb_chunks = rows_per_tile // CH
  n_groups = k_tile // grp
  mesh = plsc.VectorSubcoreMesh(core_axis_name="c", subcore_axis_name="s")

  @functools.partial(
      pl.kernel,
      out_type=jax.ShapeDtypeStruct((NC, n_pad, d2), jnp.float32),
      mesh=mesh,
      scratch_types=[
          pltpu.VMEM((k_tile, CH), jnp.int32),
          pltpu.VMEM((k_tile, CH), jnp.int32),
          [pltpu.VMEM((CH, d2), jnp.float32) for _ in range(grp)],
          pltpu.VMEM_SHARED((n_pad, d2), jnp.float32),
          pltpu.SemaphoreType.DMA,
          pltpu.SemaphoreType.DMA,
      ],
      compiler_params=pltpu.CompilerParams(use_tc_tiling_on_sc=False),
  )
  def k(table_hbm, edges_hbm, out_hbm,
        src_v, dst_v, rows, acc, gsem, ssem):
    c = lax.axis_index("c")
    s = lax.axis_index("s")
    row0 = s * rows_per_tile
    zbuf = rows[0]

    def zrow(r, carry):
      for i in range(d2 // 16):
        zbuf[r, pl.ds(i * 16, 16)] = jnp.zeros((16,), jnp.float32)
      return carry

    lax.fori_loop(0, CH, zrow, 0)
    for t in range(rb_chunks):
      pltpu.sync_copy(zbuf, acc.at[pl.ds(row0 + t * CH, CH)])
    pltpu.sync_copy(edges_hbm.at[0, pl.ds(s * k_tile, k_tile)], src_v)
    pltpu.sync_copy(edges_hbm.at[1, pl.ds(s * k_tile, k_tile)], dst_v)
    plsc.subcore_barrier()

    def outer(u, carry):
      gds, sds = [], []
      for b in range(grp):
        gds.append(
            pltpu.async_copy(table_hbm.at[c].at[src_v.at[u * grp + b]],
                             rows[b], gsem))
      for b in range(grp):
        gds[b].wait()
        sds.append(
            pltpu.async_copy(rows[b], acc.at[dst_v.at[u * grp + b]], ssem,
                             add=True))
      for b in range(grp):
        sds[b].wait()
      return carry

    lax.fori_loop(0, n_groups, outer, 0)
    plsc.subcore_barrier()

    def readback(t, carry):
      sl = pl.ds(row0 + t * CH, CH)
      pltpu.sync_copy(acc.at[sl], rows[0])
      pltpu.sync_copy(rows[0], out_hbm.at[c, sl])
      return carry

    lax.fori_loop(0, rb_chunks, readback, 0)

  return k


def _deg_sc(n_pad, k_tile):
  rows_per_tile = n_pad // NS
  k_half = k_tile // 2
  mesh = plsc.VectorSubcoreMesh(core_axis_name="c", subcore_axis_name="s")

  @functools.partial(
      pl.kernel,
      out_type=jax.ShapeDtypeStruct((NC, n_pad), jnp.float32),
      mesh=mesh,
      scratch_types=[
          pltpu.VMEM((k_half, CH), jnp.int32),
          pltpu.VMEM((CH,), jnp.float32),
          pltpu.VMEM((rows_per_tile,), jnp.float32),
          pltpu.VMEM_SHARED((n_pad,), jnp.float32),
      ],
  )
  def k(edges_hbm, out_hbm, dst_v, ones_v, rb_v, acc):
    c = lax.axis_index("c")
    s = lax.axis_index("s")
    row0 = s * rows_per_tile
    for i in range(CH // 16):
      ones_v[pl.ds(i * 16, 16)] = jnp.ones((16,), jnp.float32)

    def zrow(r, carry):
      rb_v[pl.ds(r * 16, 16)] = jnp.zeros((16,), jnp.float32)
      return carry

    lax.fori_loop(0, rows_per_tile // 16, zrow, 0)
    pltpu.sync_copy(rb_v, acc.at[pl.ds(row0, rows_per_tile)])
    pltpu.sync_copy(edges_hbm.at[1, pl.ds(s * k_tile + c * k_half, k_half)],
                    dst_v)
    plsc.subcore_barrier()

    def body(j, carry):
      pltpu.sync_copy(ones_v, acc.at[dst_v.at[j]], add=True)
      return carry

    lax.fori_loop(0, k_half, body, 0)
    plsc.subcore_barrier()
    sl = pl.ds(row0, rows_per_tile)
    pltpu.sync_copy(acc.at[sl], rb_v)
    pltpu.sync_copy(rb_v, out_hbm.at[c, sl])

  return k


def _dis_from_parts(deg_p):
  deg = deg_p[0] + deg_p[1] + 1.0
  return lax.rsqrt(deg)


def _edge_prep_body(n, k_rows, ei_ref, out_ref):
  ei = ei_ref[...]
  pad_rows = k_rows - ei.shape[1]
  out_ref[...] = jnp.concatenate(
      [ei, jnp.full((2, pad_rows, CH), n, jnp.int32)], axis=1)


def _split_cols(x):
  d2 = x.shape[1] // 2
  return jnp.stack([x[:, :d2], x[:, d2:]])


def _cat_cols(ref):
  return jnp.concatenate([ref[0], ref[1]], axis=1)


def _tc1_body(n, deg_p_ref, x_ref, w1_ref, hs1_ref):
  n_pad = deg_p_ref.shape[1]
  dis = _dis_from_parts(deg_p_ref[...])[:n]
  h = jnp.dot(x_ref[...], w1_ref[...], preferred_element_type=jnp.float32)
  hs = h * dis[:, None]
  hs = jnp.concatenate(
      [hs, jnp.zeros((n_pad - n, hs.shape[1]), jnp.float32)], axis=0)
  hs1_ref[...] = _split_cols(hs)


def _tc2_body(n, deg_p_ref, seg_ref, hs1_ref, b1_ref, w2_ref, hs2_ref):
  dis = _dis_from_parts(deg_p_ref[...])
  agg = (_cat_cols(seg_ref) + _cat_cols(hs1_ref)) * dis[:, None] + b1_ref[...]
  h = jnp.maximum(agg, 0.0)
  n_pad = h.shape[0]
  valid = lax.broadcasted_iota(jnp.int32, (n_pad, 1), 0) < n
  h = jnp.where(valid, h, 0.0)
  hs2 = jnp.dot(h, w2_ref[...], preferred_element_type=jnp.float32)
  hs2_ref[...] = _split_cols(hs2 * dis[:, None])


def _tc3_body(n, deg_p_ref, seg_ref, hs2_ref, b2_ref, out_ref):
  dis = _dis_from_parts(deg_p_ref[...])[:n]
  agg = (_cat_cols(seg_ref) + _cat_cols(hs2_ref))[:n]
  out_ref[...] = agg * dis[:, None] + b2_ref[...]


def kernel(x, edge_index, W1, b1, W2, b2):
  n, d_in = x.shape
  d_hid = W1.shape[1]
  d_out = W2.shape[1]
  e = edge_index.shape[1]

  n_pad = ((n + NS * CH) // (NS * CH)) * NS * CH
  epc = NS * CH
  k_tile = (e + epc - 1) // epc
  k_tile = ((k_tile + 7) // 8) * 8
  e_pad = k_tile * epc

  k_rows = NS * k_tile
  assert e % CH == 0
  edges = pl.pallas_call(
      functools.partial(_edge_prep_body, n, k_rows),
      out_shape=jax.ShapeDtypeStruct((2, k_rows, CH), jnp.int32),
  )(edge_index.reshape(2, e // CH, CH))

  deg_p = _deg_sc(n_pad, k_tile)(edges)

  hs1 = pl.pallas_call(
      functools.partial(_tc1_body, n),
      out_shape=jax.ShapeDtypeStruct((NC, n_pad, d_hid // 2), jnp.float32),
  )(deg_p, x, W1)

  seg1 = _segsum_sc(n_pad, d_hid // 2, k_tile, 4)(hs1, edges)

  hs2 = pl.pallas_call(
      functools.partial(_tc2_body, n),
      out_shape=jax.ShapeDtypeStruct((NC, n_pad, d_out // 2), jnp.float32),
  )(deg_p, seg1, hs1, b1, W2)

  seg2 = _segsum_sc(n_pad, d_out // 2, k_tile, 4)(hs2, edges)

  return pl.pallas_call(
      functools.partial(_tc3_body, n),
      out_shape=jax.ShapeDtypeStruct((n, d_out), jnp.float32),
  )(deg_p, seg2, hs2, b2)


# speedup vs baseline: 1.3157x; 1.3157x over previous
import functools

import jax
import jax.numpy as jnp
from jax import lax
from jax.experimental import pallas as pl
from jax.experimental.pallas import tpu as pltpu
from jax.experimental.pallas import tpu_sc as plsc

NC = 2
NS = 16
CH = 128


def _segsum_sc(n_pad, d2, k_tile, grp):
  rows_per_tile = n_pad // NS
  rb_chunks = rows_per_tile // CH
  n_groups = k_tile // grp
  mesh = plsc.VectorSubcoreMesh(core_axis_name="c", subcore_axis_name="s")

  @functools.partial(
      pl.kernel,
      out_type=jax.ShapeDtypeStruct((NC, n_pad, d2), jnp.float32),
      mesh=mesh,
      scratch_types=[
          pltpu.VMEM((k_tile, CH), jnp.int32),
          pltpu.VMEM((k_tile, CH), jnp.int32),
          [pltpu.VMEM((CH, d2), jnp.float32) for _ in range(grp)],
          pltpu.VMEM_SHARED((n_pad, d2), jnp.float32),
          pltpu.SemaphoreType.DMA,
          pltpu.SemaphoreType.DMA,
      ],
      compiler_params=pltpu.CompilerParams(use_tc_tiling_on_sc=False),
  )
  def k(table_hbm, edges_hbm, out_hbm,
        src_v, dst_v, rows, acc, gsem, ssem):
    c = lax.axis_index("c")
    s = lax.axis_index("s")
    row0 = s * rows_per_tile
    zbuf = rows[0]

    def zrow(r, carry):
      for i in range(d2 // 16):
        zbuf[r, pl.ds(i * 16, 16)] = jnp.zeros((16,), jnp.float32)
      return carry

    lax.fori_loop(0, CH, zrow, 0)
    for t in range(rb_chunks):
      pltpu.sync_copy(zbuf, acc.at[pl.ds(row0 + t * CH, CH)])
    pltpu.sync_copy(edges_hbm.at[0, pl.ds(s * k_tile, k_tile)], src_v)
    pltpu.sync_copy(edges_hbm.at[1, pl.ds(s * k_tile, k_tile)], dst_v)
    plsc.subcore_barrier()

    def outer(u, carry):
      gds, sds = [], []
      for b in range(grp):
        gds.append(
            pltpu.async_copy(table_hbm.at[c].at[src_v.at[u * grp + b]],
                             rows[b], gsem))
      for b in range(grp):
        gds[b].wait()
        sds.append(
            pltpu.async_copy(rows[b], acc.at[dst_v.at[u * grp + b]], ssem,
                             add=True))
      for b in range(grp):
        sds[b].wait()
      return carry

    lax.fori_loop(0, n_groups, outer, 0)
    plsc.subcore_barrier()

    def readback(t, carry):
      sl = pl.ds(row0 + t * CH, CH)
      pltpu.sync_copy(acc.at[sl], rows[0])
      pltpu.sync_copy(rows[0], out_hbm.at[c, sl])
      return carry

    lax.fori_loop(0, rb_chunks, readback, 0)

  return k


def _deg_sc(n_pad, k_tile):
  rows_per_tile = n_pad // NS
  k_half = k_tile // 2
  mesh = plsc.VectorSubcoreMesh(core_axis_name="c", subcore_axis_name="s")

  @functools.partial(
      pl.kernel,
      out_type=jax.ShapeDtypeStruct((NC, n_pad), jnp.float32),
      mesh=mesh,
      scratch_types=[
          pltpu.VMEM((k_half, CH), jnp.int32),
          pltpu.VMEM((CH,), jnp.float32),
          pltpu.VMEM((rows_per_tile,), jnp.float32),
          pltpu.VMEM_SHARED((n_pad,), jnp.float32),
      ],
  )
  def k(edges_hbm, out_hbm, dst_v, ones_v, rb_v, acc):
    c = lax.axis_index("c")
    s = lax.axis_index("s")
    row0 = s * rows_per_tile
    for i in range(CH // 16):
      ones_v[pl.ds(i * 16, 16)] = jnp.ones((16,), jnp.float32)

    def zrow(r, carry):
      rb_v[pl.ds(r * 16, 16)] = jnp.zeros((16,), jnp.float32)
      return carry

    lax.fori_loop(0, rows_per_tile // 16, zrow, 0)
    pltpu.sync_copy(rb_v, acc.at[pl.ds(row0, rows_per_tile)])
    pltpu.sync_copy(edges_hbm.at[1, pl.ds(s * k_tile + c * k_half, k_half)],
                    dst_v)
    plsc.subcore_barrier()

    def body(j, carry):
      pltpu.sync_copy(ones_v, acc.at[dst_v.at[j]], add=True)
      return carry

    lax.fori_loop(0, k_half, body, 0)
    plsc.subcore_barrier()
    sl = pl.ds(row0, rows_per_tile)
    pltpu.sync_copy(acc.at[sl], rb_v)
    pltpu.sync_copy(rb_v, out_hbm.at[c, sl])

  return k


def _dis_from_parts(deg_p):
  deg = deg_p[0] + deg_p[1] + 1.0
  return lax.rsqrt(deg)


def _edge_prep_body(n, k_rows, ei_ref, out_ref):
  ei = ei_ref[...]
  pad_rows = k_rows - ei.shape[1]
  out_ref[...] = jnp.concatenate(
      [ei, jnp.full((2, pad_rows, CH), n, jnp.int32)], axis=1)


def _split_cols(x):
  d2 = x.shape[1] // 2
  return jnp.stack([x[:, :d2], x[:, d2:]])


def _cat_cols(ref):
  return jnp.concatenate([ref[0], ref[1]], axis=1)


def _tc1_body(n, deg_p_ref, x_ref, w1_ref, hs1_ref):
  n_pad = deg_p_ref.shape[1]
  dis = _dis_from_parts(deg_p_ref[...])[:n]
  h = jnp.dot(x_ref[...], w1_ref[...], preferred_element_type=jnp.float32)
  hs = h * dis[:, None]
  hs = jnp.concatenate(
      [hs, jnp.zeros((n_pad - n, hs.shape[1]), jnp.float32)], axis=0)
  hs1_ref[...] = _split_cols(hs)


def _tc2_body(n, deg_p_ref, seg_ref, hs1_ref, b1_ref, t2_ref):
  dis = _dis_from_parts(deg_p_ref[...])
  agg = (_cat_cols(seg_ref) + _cat_cols(hs1_ref)) * dis[:, None] + b1_ref[...]
  h = jnp.maximum(agg, 0.0)
  n_pad = h.shape[0]
  valid = lax.broadcasted_iota(jnp.int32, (n_pad, 1), 0) < n
  h = jnp.where(valid, h, 0.0)
  t2_ref[...] = _split_cols(h * dis[:, None])


def _tc3_body(n, deg_p_ref, seg_ref, t2_ref, w2_ref, b2_ref, out_ref):
  dis = _dis_from_parts(deg_p_ref[...])[:n]
  agg = (_cat_cols(seg_ref) + _cat_cols(t2_ref))[:n] * dis[:, None]
  out_ref[...] = jnp.dot(
      agg, w2_ref[...], preferred_element_type=jnp.float32) + b2_ref[...]


def kernel(x, edge_index, W1, b1, W2, b2):
  n, d_in = x.shape
  d_hid = W1.shape[1]
  d_out = W2.shape[1]
  e = edge_index.shape[1]

  n_pad = ((n + NS * CH) // (NS * CH)) * NS * CH
  epc = NS * CH
  k_tile = (e + epc - 1) // epc
  k_tile = ((k_tile + 7) // 8) * 8
  e_pad = k_tile * epc

  k_rows = NS * k_tile
  assert e % CH == 0
  edges = pl.pallas_call(
      functools.partial(_edge_prep_body, n, k_rows),
      out_shape=jax.ShapeDtypeStruct((2, k_rows, CH), jnp.int32),
  )(edge_index.reshape(2, e // CH, CH))

  deg_p = _deg_sc(n_pad, k_tile)(edges)

  hs1 = pl.pallas_call(
      functools.partial(_tc1_body, n),
      out_shape=jax.ShapeDtypeStruct((NC, n_pad, d_hid // 2), jnp.float32),
  )(deg_p, x, W1)

  seg1 = _segsum_sc(n_pad, d_hid // 2, k_tile, 4)(hs1, edges)

  t2 = pl.pallas_call(
      functools.partial(_tc2_body, n),
      out_shape=jax.ShapeDtypeStruct((NC, n_pad, d_hid // 2), jnp.float32),
  )(deg_p, seg1, hs1, b1)

  seg2 = _segsum_sc(n_pad, d_hid // 2, k_tile, 4)(t2, edges)

  return pl.pallas_call(
      functools.partial(_tc3_body, n),
      out_shape=jax.ShapeDtypeStruct((n, d_out), jnp.float32),
  )(deg_p, seg2, t2, W2, b2)


# speedup vs baseline: 1.3336x; 1.0136x over previous
import functools

import jax
import jax.numpy as jnp
from jax import lax
from jax.experimental import pallas as pl
from jax.experimental.pallas import tpu as pltpu
from jax.experimental.pallas import tpu_sc as plsc

NC = 2
NS = 16
CH = 128


def _segsum_sc(n_pad, d2, k_tile, grp):
  rows_per_tile = n_pad // NS
  rb_chunks = rows_per_tile // CH
  n_groups = k_tile // grp
  mesh = plsc.VectorSubcoreMesh(core_axis_name="c", subcore_axis_name="s")

  @functools.partial(
      pl.kernel,
      out_type=jax.ShapeDtypeStruct((NC, n_pad, d2), jnp.float32),
      mesh=mesh,
      scratch_types=[
          pltpu.VMEM((k_tile, CH), jnp.int32),
          pltpu.VMEM((k_tile, CH), jnp.int32),
          [pltpu.VMEM((CH, d2), jnp.float32) for _ in range(grp)],
          pltpu.VMEM_SHARED((n_pad, d2), jnp.float32),
          pltpu.SemaphoreType.DMA,
          pltpu.SemaphoreType.DMA,
      ],
      compiler_params=pltpu.CompilerParams(use_tc_tiling_on_sc=False),
  )
  def k(table_hbm, edges_hbm, out_hbm,
        src_v, dst_v, rows, acc, gsem, ssem):
    c = lax.axis_index("c")
    s = lax.axis_index("s")
    row0 = s * rows_per_tile
    zbuf = rows[0]

    def zrow(r, carry):
      for i in range(d2 // 16):
        zbuf[r, pl.ds(i * 16, 16)] = jnp.zeros((16,), jnp.float32)
      return carry

    i0 = pltpu.async_copy(edges_hbm.at[0, pl.ds(s * k_tile, k_tile)], src_v,
                          gsem)
    i1 = pltpu.async_copy(edges_hbm.at[1, pl.ds(s * k_tile, k_tile)], dst_v,
                          ssem)
    lax.fori_loop(0, CH, zrow, 0)
    for t in range(rb_chunks):
      pltpu.sync_copy(zbuf, acc.at[pl.ds(row0 + t * CH, CH)])
    i0.wait()
    i1.wait()
    plsc.subcore_barrier()

    def outer(u, carry):
      gds, sds = [], []
      for b in range(grp):
        gds.append(
            pltpu.async_copy(table_hbm.at[c].at[src_v.at[u * grp + b]],
                             rows[b], gsem))
      for b in range(grp):
        gds[b].wait()
        sds.append(
            pltpu.async_copy(rows[b], acc.at[dst_v.at[u * grp + b]], ssem,
                             add=True))
      for b in range(grp):
        sds[b].wait()
      return carry

    lax.fori_loop(0, n_groups, outer, 0)
    plsc.subcore_barrier()

    pltpu.sync_copy(acc.at[pl.ds(row0, rows_per_tile)],
                    out_hbm.at[c, pl.ds(row0, rows_per_tile)])

  return k


def _deg_sc(n_pad, k_tile):
  rows_per_tile = n_pad // NS
  k_half = k_tile // 2
  mesh = plsc.VectorSubcoreMesh(core_axis_name="c", subcore_axis_name="s")

  @functools.partial(
      pl.kernel,
      out_type=jax.ShapeDtypeStruct((NC, n_pad), jnp.float32),
      mesh=mesh,
      scratch_types=[
          pltpu.VMEM((k_half, CH), jnp.int32),
          pltpu.VMEM((CH,), jnp.float32),
          pltpu.VMEM((rows_per_tile,), jnp.float32),
          pltpu.VMEM_SHARED((n_pad,), jnp.float32),
      ],
  )
  def k(edges_hbm, out_hbm, dst_v, ones_v, rb_v, acc):
    c = lax.axis_index("c")
    s = lax.axis_index("s")
    row0 = s * rows_per_tile
    for i in range(CH // 16):
      ones_v[pl.ds(i * 16, 16)] = jnp.ones((16,), jnp.float32)

    def zrow(r, carry):
      rb_v[pl.ds(r * 16, 16)] = jnp.zeros((16,), jnp.float32)
      return carry

    lax.fori_loop(0, rows_per_tile // 16, zrow, 0)
    pltpu.sync_copy(rb_v, acc.at[pl.ds(row0, rows_per_tile)])
    pltpu.sync_copy(edges_hbm.at[1, pl.ds(s * k_tile + c * k_half, k_half)],
                    dst_v)
    plsc.subcore_barrier()

    def body(j, carry):
      pltpu.sync_copy(ones_v, acc.at[dst_v.at[j]], add=True)
      return carry

    lax.fori_loop(0, k_half, body, 0)
    plsc.subcore_barrier()
    sl = pl.ds(row0, rows_per_tile)
    pltpu.sync_copy(acc.at[sl], rb_v)
    pltpu.sync_copy(rb_v, out_hbm.at[c, sl])

  return k


def _dis_from_parts(deg_p):
  deg = deg_p[0] + deg_p[1] + 1.0
  return lax.rsqrt(deg)


def _edge_prep_body(n, k_rows, ei_ref, out_ref):
  ei = ei_ref[...]
  pad_rows = k_rows - ei.shape[1]
  out_ref[...] = jnp.concatenate(
      [ei, jnp.full((2, pad_rows, CH), n, jnp.int32)], axis=1)


def _split_cols(x):
  d2 = x.shape[1] // 2
  return jnp.stack([x[:, :d2], x[:, d2:]])


def _cat_cols(ref):
  return jnp.concatenate([ref[0], ref[1]], axis=1)


def _tc1_body(n, deg_p_ref, x_ref, w1_ref, hs1_ref):
  n_pad = deg_p_ref.shape[1]
  dis = _dis_from_parts(deg_p_ref[...])[:n]
  h = jnp.dot(x_ref[...], w1_ref[...], preferred_element_type=jnp.float32)
  hs = h * dis[:, None]
  hs = jnp.concatenate(
      [hs, jnp.zeros((n_pad - n, hs.shape[1]), jnp.float32)], axis=0)
  hs1_ref[...] = _split_cols(hs)


def _tc2_body(n, deg_p_ref, seg_ref, hs1_ref, b1_ref, t2_ref):
  dis = _dis_from_parts(deg_p_ref[...])
  agg = (_cat_cols(seg_ref) + _cat_cols(hs1_ref)) * dis[:, None] + b1_ref[...]
  h = jnp.maximum(agg, 0.0)
  n_pad = h.shape[0]
  valid = lax.broadcasted_iota(jnp.int32, (n_pad, 1), 0) < n
  h = jnp.where(valid, h, 0.0)
  t2_ref[...] = _split_cols(h * dis[:, None])


def _tc3_body(n, deg_p_ref, seg_ref, t2_ref, w2_ref, b2_ref, out_ref):
  dis = _dis_from_parts(deg_p_ref[...])[:n]
  agg = (_cat_cols(seg_ref) + _cat_cols(t2_ref))[:n] * dis[:, None]
  out_ref[...] = jnp.dot(
      agg, w2_ref[...], preferred_element_type=jnp.float32) + b2_ref[...]


def kernel(x, edge_index, W1, b1, W2, b2):
  n, d_in = x.shape
  d_hid = W1.shape[1]
  d_out = W2.shape[1]
  e = edge_index.shape[1]

  n_pad = ((n + NS * CH) // (NS * CH)) * NS * CH
  epc = NS * CH
  k_tile = (e + epc - 1) // epc
  k_tile = ((k_tile + 7) // 8) * 8
  e_pad = k_tile * epc

  k_rows = NS * k_tile
  assert e % CH == 0
  edges = pl.pallas_call(
      functools.partial(_edge_prep_body, n, k_rows),
      out_shape=jax.ShapeDtypeStruct((2, k_rows, CH), jnp.int32),
  )(edge_index.reshape(2, e // CH, CH))

  deg_p = _deg_sc(n_pad, k_tile)(edges)

  hs1 = pl.pallas_call(
      functools.partial(_tc1_body, n),
      out_shape=jax.ShapeDtypeStruct((NC, n_pad, d_hid // 2), jnp.float32),
  )(deg_p, x, W1)

  seg1 = _segsum_sc(n_pad, d_hid // 2, k_tile, 4)(hs1, edges)

  t2 = pl.pallas_call(
      functools.partial(_tc2_body, n),
      out_shape=jax.ShapeDtypeStruct((NC, n_pad, d_hid // 2), jnp.float32),
  )(deg_p, seg1, hs1, b1)

  seg2 = _segsum_sc(n_pad, d_hid // 2, k_tile, 4)(t2, edges)

  return pl.pallas_call(
      functools.partial(_tc3_body, n),
      out_shape=jax.ShapeDtypeStruct((n, d_out), jnp.float32),
  )(deg_p, seg2, t2, W2, b2)


# speedup vs baseline: 1.3453x; 1.0088x over previous
import functools

import jax
import jax.numpy as jnp
from jax import lax
from jax.experimental import pallas as pl
from jax.experimental.pallas import tpu as pltpu
from jax.experimental.pallas import tpu_sc as plsc

NC = 2
NS = 16
CH = 128


def _segsum_sc(n_pad, d2, k_tile, grp):
  rows_per_tile = n_pad // NS
  rb_chunks = rows_per_tile // CH
  n_groups = k_tile // grp
  mesh = plsc.VectorSubcoreMesh(core_axis_name="c", subcore_axis_name="s")

  @functools.partial(
      pl.kernel,
      out_type=jax.ShapeDtypeStruct((NC, n_pad, d2), jnp.float32),
      mesh=mesh,
      scratch_types=[
          pltpu.VMEM((k_tile, CH), jnp.int32),
          pltpu.VMEM((k_tile, CH), jnp.int32),
          [pltpu.VMEM((CH, d2), jnp.float32) for _ in range(grp)],
          pltpu.VMEM_SHARED((n_pad, d2), jnp.float32),
          pltpu.SemaphoreType.DMA,
          pltpu.SemaphoreType.DMA,
      ],
      compiler_params=pltpu.CompilerParams(use_tc_tiling_on_sc=False),
  )
  def k(table_hbm, edges_hbm, out_hbm,
        src_v, dst_v, rows, acc, gsem, ssem):
    c = lax.axis_index("c")
    s = lax.axis_index("s")
    row0 = s * rows_per_tile
    zbuf = rows[0]

    def zrow(r, carry):
      for i in range(d2 // 16):
        zbuf[r, pl.ds(i * 16, 16)] = jnp.zeros((16,), jnp.float32)
      return carry

    i0 = pltpu.async_copy(edges_hbm.at[0, pl.ds(s * k_tile, k_tile)], src_v,
                          gsem)
    i1 = pltpu.async_copy(edges_hbm.at[1, pl.ds(s * k_tile, k_tile)], dst_v,
                          ssem)
    lax.fori_loop(0, CH, zrow, 0)
    for t in range(rb_chunks):
      pltpu.sync_copy(zbuf, acc.at[pl.ds(row0 + t * CH, CH)])
    i0.wait()
    i1.wait()
    plsc.subcore_barrier()

    def outer(u, carry):
      gds, sds = [], []
      for b in range(grp):
        gds.append(
            pltpu.async_copy(table_hbm.at[c].at[src_v.at[u * grp + b]],
                             rows[b], gsem))
      for b in range(grp):
        gds[b].wait()
        sds.append(
            pltpu.async_copy(rows[b], acc.at[dst_v.at[u * grp + b]], ssem,
                             add=True))
      for b in range(grp):
        sds[b].wait()
      return carry

    lax.fori_loop(0, n_groups, outer, 0)
    plsc.subcore_barrier()

    pltpu.sync_copy(acc.at[pl.ds(row0, rows_per_tile)],
                    out_hbm.at[c, pl.ds(row0, rows_per_tile)])

  return k


def _deg_sc(n_pad, e_chunks, kd):
  rows_per_tile = n_pad // NS
  last_cnt = e_chunks - (NC * NS - 1) * kd
  assert 0 < last_cnt <= kd
  mesh = plsc.VectorSubcoreMesh(core_axis_name="c", subcore_axis_name="s")

  @functools.partial(
      pl.kernel,
      out_type=jax.ShapeDtypeStruct((NC, n_pad), jnp.float32),
      mesh=mesh,
      scratch_types=[
          pltpu.VMEM((kd, CH), jnp.int32),
          pltpu.VMEM((CH,), jnp.float32),
          pltpu.VMEM((rows_per_tile,), jnp.float32),
          pltpu.VMEM_SHARED((n_pad,), jnp.float32),
      ],
      compiler_params=pltpu.CompilerParams(use_tc_tiling_on_sc=False),
  )
  def k(edges_hbm, out_hbm, dst_v, ones_v, rb_v, acc):
    c = lax.axis_index("c")
    s = lax.axis_index("s")
    row0 = s * rows_per_tile
    w = c * NS + s
    is_last = w == NC * NS - 1
    n_my = jnp.where(is_last, last_cnt, kd)
    for i in range(CH // 16):
      ones_v[pl.ds(i * 16, 16)] = jnp.ones((16,), jnp.float32)

    def zrow(r, carry):
      rb_v[pl.ds(r * 16, 16)] = jnp.zeros((16,), jnp.float32)
      return carry

    lax.fori_loop(0, rows_per_tile // 16, zrow, 0)
    pltpu.sync_copy(rb_v, acc.at[pl.ds(row0, rows_per_tile)])

    @pl.when(jnp.logical_not(is_last))
    def _():
      pltpu.sync_copy(edges_hbm.at[1, pl.ds(w * kd, kd)], dst_v)

    @pl.when(is_last)
    def _():
      pltpu.sync_copy(edges_hbm.at[1, pl.ds(w * kd, last_cnt)],
                      dst_v.at[pl.ds(0, last_cnt)])

    plsc.subcore_barrier()

    def body(j, carry):
      pltpu.sync_copy(ones_v, acc.at[dst_v.at[j]], add=True)
      return carry

    lax.fori_loop(0, n_my, body, 0)
    plsc.subcore_barrier()
    sl = pl.ds(row0, rows_per_tile)
    pltpu.sync_copy(acc.at[sl], rb_v)
    pltpu.sync_copy(rb_v, out_hbm.at[c, sl])

  return k


def _dis_from_parts(deg_p):
  deg = deg_p[0] + deg_p[1] + 1.0
  return lax.rsqrt(deg)


def _tc_a_body(n, k_rows, ei_ref, x_ref, w1_ref, edges_ref, h1_ref):
  ei = ei_ref[...]
  pad_rows = k_rows - ei.shape[1]
  edges_ref[...] = jnp.concatenate(
      [ei, jnp.full((2, pad_rows, CH), n, jnp.int32)], axis=1)
  h1_ref[...] = jnp.dot(x_ref[...], w1_ref[...],
                        preferred_element_type=jnp.float32)


def _split_cols(x):
  d2 = x.shape[1] // 2
  return jnp.stack([x[:, :d2], x[:, d2:]])


def _cat_cols(ref):
  return jnp.concatenate([ref[0], ref[1]], axis=1)


def _tc1_body(n, deg_p_ref, h1_ref, hs1_ref):
  n_pad = deg_p_ref.shape[1]
  dis = _dis_from_parts(deg_p_ref[...])[:n]
  hs = h1_ref[...] * dis[:, None]
  hs = jnp.concatenate(
      [hs, jnp.zeros((n_pad - n, hs.shape[1]), jnp.float32)], axis=0)
  hs1_ref[...] = _split_cols(hs)


def _tc2_body(n, deg_p_ref, seg_ref, hs1_ref, b1_ref, t2_ref):
  dis = _dis_from_parts(deg_p_ref[...])
  agg = (_cat_cols(seg_ref) + _cat_cols(hs1_ref)) * dis[:, None] + b1_ref[...]
  h = jnp.maximum(agg, 0.0)
  n_pad = h.shape[0]
  valid = lax.broadcasted_iota(jnp.int32, (n_pad, 1), 0) < n
  h = jnp.where(valid, h, 0.0)
  t2_ref[...] = _split_cols(h * dis[:, None])


def _tc3_body(n, deg_p_ref, seg_ref, t2_ref, w2_ref, b2_ref, out_ref):
  dis = _dis_from_parts(deg_p_ref[...])[:n]
  agg = (_cat_cols(seg_ref) + _cat_cols(t2_ref))[:n] * dis[:, None]
  out_ref[...] = jnp.dot(
      agg, w2_ref[...], preferred_element_type=jnp.float32) + b2_ref[...]


def kernel(x, edge_index, W1, b1, W2, b2):
  n, d_in = x.shape
  d_hid = W1.shape[1]
  d_out = W2.shape[1]
  e = edge_index.shape[1]

  n_pad = ((n + NS * CH) // (NS * CH)) * NS * CH
  epc = NS * CH
  k_tile = (e + epc - 1) // epc
  k_tile = ((k_tile + 7) // 8) * 8
  e_pad = k_tile * epc

  k_rows = NS * k_tile
  assert e % CH == 0
  e_chunks = e // CH
  ei3 = edge_index.reshape(2, e_chunks, CH)

  kd = ((e_chunks + NC * NS - 1) // (NC * NS) + 7) // 8 * 8
  deg_p = _deg_sc(n_pad, e_chunks, kd)(ei3)

  edges, h1 = pl.pallas_call(
      functools.partial(_tc_a_body, n, k_rows),
      out_shape=(jax.ShapeDtypeStruct((2, k_rows, CH), jnp.int32),
                 jax.ShapeDtypeStruct((n, d_hid), jnp.float32)),
  )(ei3, x, W1)

  hs1 = pl.pallas_call(
      functools.partial(_tc1_body, n),
      out_shape=jax.ShapeDtypeStruct((NC, n_pad, d_hid // 2), jnp.float32),
  )(deg_p, h1)

  seg1 = _segsum_sc(n_pad, d_hid // 2, k_tile, 4)(hs1, edges)

  t2 = pl.pallas_call(
      functools.partial(_tc2_body, n),
      out_shape=jax.ShapeDtypeStruct((NC, n_pad, d_hid // 2), jnp.float32),
  )(deg_p, seg1, hs1, b1)

  seg2 = _segsum_sc(n_pad, d_hid // 2, k_tile, 4)(t2, edges)

  return pl.pallas_call(
      functools.partial(_tc3_body, n),
      out_shape=jax.ShapeDtypeStruct((n, d_out), jnp.float32),
  )(deg_p, seg2, t2, W2, b2)


# speedup vs baseline: 1.4235x; 1.0581x over previous
import functools

import jax
import jax.numpy as jnp
from jax import lax
from jax.experimental import pallas as pl
from jax.experimental.pallas import tpu as pltpu
from jax.experimental.pallas import tpu_sc as plsc

NC = 2
NS = 16
CH = 128


def _segsum_sc(n_pad, d2, k_tile, grp):
  rows_per_tile = n_pad // NS
  rb_chunks = rows_per_tile // CH
  n_groups = k_tile // grp
  mesh = plsc.VectorSubcoreMesh(core_axis_name="c", subcore_axis_name="s")

  @functools.partial(
      pl.kernel,
      out_type=jax.ShapeDtypeStruct((NC, n_pad, d2), jnp.float32),
      mesh=mesh,
      scratch_types=[
          pltpu.VMEM((k_tile, CH), jnp.int32),
          pltpu.VMEM((k_tile, CH), jnp.int32),
          [pltpu.VMEM((CH, d2), jnp.float32) for _ in range(grp)],
          pltpu.VMEM_SHARED((n_pad, d2), jnp.float32),
          pltpu.SemaphoreType.DMA,
          pltpu.SemaphoreType.DMA,
      ],
      compiler_params=pltpu.CompilerParams(use_tc_tiling_on_sc=False),
  )
  def k(table_hbm, edges_hbm, out_hbm,
        src_v, dst_v, rows, acc, gsem, ssem):
    c = lax.axis_index("c")
    s = lax.axis_index("s")
    row0 = s * rows_per_tile
    zbuf = rows[0]

    def zrow(r, carry):
      for i in range(d2 // 16):
        zbuf[r, pl.ds(i * 16, 16)] = jnp.zeros((16,), jnp.float32)
      return carry

    i0 = pltpu.async_copy(edges_hbm.at[0, pl.ds(s * k_tile, k_tile)], src_v,
                          gsem)
    i1 = pltpu.async_copy(edges_hbm.at[1, pl.ds(s * k_tile, k_tile)], dst_v,
                          ssem)
    lax.fori_loop(0, CH, zrow, 0)
    for t in range(rb_chunks):
      pltpu.sync_copy(zbuf, acc.at[pl.ds(row0 + t * CH, CH)])
    i0.wait()
    i1.wait()
    plsc.subcore_barrier()

    def outer(u, carry):
      gds, sds = [], []
      for b in range(grp):
        gds.append(
            pltpu.async_copy(table_hbm.at[c].at[src_v.at[u * grp + b]],
                             rows[b], gsem))
      for b in range(grp):
        gds[b].wait()
        sds.append(
            pltpu.async_copy(rows[b], acc.at[dst_v.at[u * grp + b]], ssem,
                             add=True))
      for b in range(grp):
        sds[b].wait()
      return carry

    lax.fori_loop(0, n_groups, outer, 0)
    plsc.subcore_barrier()

    pltpu.sync_copy(acc.at[pl.ds(row0, rows_per_tile)],
                    out_hbm.at[c, pl.ds(row0, rows_per_tile)])

  return k


def _gcn_sc(n_pad, d2, k_tile, grp, n):
  rows_per_tile = n_pad // NS
  rb_chunks = rows_per_tile // CH
  n_groups = k_tile // grp
  mesh = plsc.VectorSubcoreMesh(core_axis_name="c", subcore_axis_name="s")

  @functools.partial(
      pl.kernel,
      out_type=(jax.ShapeDtypeStruct((NC, n_pad, d2), jnp.float32),
                jax.ShapeDtypeStruct((NC, n_pad, d2), jnp.float32)),
      mesh=mesh,
      scratch_types=[
          pltpu.VMEM((k_tile, CH), jnp.int32),
          pltpu.VMEM((k_tile, CH), jnp.int32),
          [pltpu.VMEM((CH, d2), jnp.float32) for _ in range(grp)],
          pltpu.VMEM((CH, d2), jnp.float32),
          pltpu.VMEM((CH, d2), jnp.float32),
          pltpu.VMEM((CH, d2), jnp.float32),
          pltpu.VMEM((d2,), jnp.float32),
          pltpu.VMEM_SHARED((n_pad, d2), jnp.float32),
          pltpu.SemaphoreType.DMA,
          pltpu.SemaphoreType.DMA,
      ],
      compiler_params=pltpu.CompilerParams(use_tc_tiling_on_sc=False),
  )
  def k(hs1_hbm, edges_hbm, dis_hbm, b1_hbm, t2_hbm, seg2_hbm,
        src_v, dst_v, rows, wbuf, hbuf, dbuf, b1v, acc, gsem, ssem):
    c = lax.axis_index("c")
    s = lax.axis_index("s")
    row0 = s * rows_per_tile
    rsl = pl.ds(row0, rows_per_tile)
    zbuf = rows[0]

    def zrow(r, carry):
      for i in range(d2 // 16):
        zbuf[r, pl.ds(i * 16, 16)] = jnp.zeros((16,), jnp.float32)
      return carry

    def zero_acc():
      for t in range(rb_chunks):
        pltpu.sync_copy(zbuf, acc.at[pl.ds(row0 + t * CH, CH)])

    def segsum(table_hbm, sem):
      def outer(u, carry):
        gds = []
        for b in range(grp):
          gds.append(
              pltpu.async_copy(table_hbm.at[c].at[src_v.at[u * grp + b]],
                               rows[b], sem))
        for b in range(grp):
          gds[b].wait()
          pltpu.sync_copy(rows[b], acc.at[dst_v.at[u * grp + b]], add=True)
        return carry

      lax.fori_loop(0, n_groups, outer, 0)

    i0 = pltpu.async_copy(edges_hbm.at[0, pl.ds(s * k_tile, k_tile)], src_v,
                          gsem)
    i1 = pltpu.async_copy(edges_hbm.at[1, pl.ds(s * k_tile, k_tile)], dst_v,
                          ssem)
    lax.fori_loop(0, CH, zrow, 0)
    zero_acc()
    pltpu.sync_copy(b1_hbm.at[pl.ds(c * d2, d2)], b1v)
    i0.wait()
    i1.wait()
    plsc.subcore_barrier()

    segsum(hs1_hbm, gsem)
    plsc.subcore_barrier()

    def brow(r, carry):
      for i in range(d2 // 16):
        csl = pl.ds(i * 16, 16)
        d = dbuf[r, csl]
        agg = (wbuf[r, csl] + hbuf[r, csl]) * d + b1v[csl]
        wbuf[r, csl] = jnp.maximum(agg, 0.0) * d
      return carry

    for t in range(rb_chunks):
      csl = pl.ds(row0 + t * CH, CH)
      pltpu.sync_copy(acc.at[csl], wbuf)
      pltpu.sync_copy(hs1_hbm.at[c, csl], hbuf)
      pltpu.sync_copy(dis_hbm.at[csl], dbuf)
      lax.fori_loop(0, CH, brow, 0)
      pltpu.sync_copy(wbuf, t2_hbm.at[c, csl])
    plsc.subcore_barrier()

    lax.fori_loop(0, CH, zrow, 0)
    zero_acc()
    plsc.subcore_barrier()
    segsum(t2_hbm, ssem)
    plsc.subcore_barrier()
    pltpu.sync_copy(acc.at[rsl], seg2_hbm.at[c, rsl])

  return k


def _deg_sc(n_pad, e_chunks, kd):
  rows_per_tile = n_pad // NS
  last_cnt = e_chunks - (NC * NS - 1) * kd
  assert 0 < last_cnt <= kd
  mesh = plsc.VectorSubcoreMesh(core_axis_name="c", subcore_axis_name="s")

  @functools.partial(
      pl.kernel,
      out_type=jax.ShapeDtypeStruct((NC, n_pad), jnp.float32),
      mesh=mesh,
      scratch_types=[
          pltpu.VMEM((kd, CH), jnp.int32),
          pltpu.VMEM((CH,), jnp.float32),
          pltpu.VMEM((rows_per_tile,), jnp.float32),
          pltpu.VMEM_SHARED((n_pad,), jnp.float32),
      ],
      compiler_params=pltpu.CompilerParams(use_tc_tiling_on_sc=False),
  )
  def k(edges_hbm, out_hbm, dst_v, ones_v, rb_v, acc):
    c = lax.axis_index("c")
    s = lax.axis_index("s")
    row0 = s * rows_per_tile
    w = c * NS + s
    is_last = w == NC * NS - 1
    n_my = jnp.where(is_last, last_cnt, kd)
    for i in range(CH // 16):
      ones_v[pl.ds(i * 16, 16)] = jnp.ones((16,), jnp.float32)

    def zrow(r, carry):
      rb_v[pl.ds(r * 16, 16)] = jnp.zeros((16,), jnp.float32)
      return carry

    lax.fori_loop(0, rows_per_tile // 16, zrow, 0)
    pltpu.sync_copy(rb_v, acc.at[pl.ds(row0, rows_per_tile)])

    @pl.when(jnp.logical_not(is_last))
    def _():
      pltpu.sync_copy(edges_hbm.at[1, pl.ds(w * kd, kd)], dst_v)

    @pl.when(is_last)
    def _():
      pltpu.sync_copy(edges_hbm.at[1, pl.ds(w * kd, last_cnt)],
                      dst_v.at[pl.ds(0, last_cnt)])

    plsc.subcore_barrier()

    def body(j, carry):
      pltpu.sync_copy(ones_v, acc.at[dst_v.at[j]], add=True)
      return carry

    lax.fori_loop(0, n_my, body, 0)
    plsc.subcore_barrier()
    sl = pl.ds(row0, rows_per_tile)
    pltpu.sync_copy(acc.at[sl], rb_v)
    pltpu.sync_copy(rb_v, out_hbm.at[c, sl])

  return k


def _dis_from_parts(deg_p):
  deg = deg_p[0] + deg_p[1] + 1.0
  return lax.rsqrt(deg)


def _tc_a_body(n, k_rows, ei_ref, x_ref, w1_ref, edges_ref, h1_ref):
  ei = ei_ref[...]
  pad_rows = k_rows - ei.shape[1]
  edges_ref[...] = jnp.concatenate(
      [ei, jnp.full((2, pad_rows, CH), n, jnp.int32)], axis=1)
  h1_ref[...] = jnp.dot(x_ref[...], w1_ref[...],
                        preferred_element_type=jnp.float32)


def _split_cols(x):
  d2 = x.shape[1] // 2
  return jnp.stack([x[:, :d2], x[:, d2:]])


def _cat_cols(ref):
  return jnp.concatenate([ref[0], ref[1]], axis=1)


def _tc1_body(n, deg_p_ref, h1_ref, hs1_ref, dis_ref):
  n_pad = deg_p_ref.shape[1]
  d2 = dis_ref.shape[1]
  dis = _dis_from_parts(deg_p_ref[...])[:n]
  hs = h1_ref[...] * dis[:, None]
  hs = jnp.concatenate(
      [hs, jnp.zeros((n_pad - n, hs.shape[1]), jnp.float32)], axis=0)
  hs1_ref[...] = _split_cols(hs)
  dis_ref[...] = jnp.concatenate(
      [jnp.broadcast_to(dis[:, None], (n, d2)),
       jnp.zeros((n_pad - n, d2), jnp.float32)], axis=0)


def _tc2_body(n, deg_p_ref, seg_ref, hs1_ref, b1_ref, t2_ref):
  dis = _dis_from_parts(deg_p_ref[...])
  agg = (_cat_cols(seg_ref) + _cat_cols(hs1_ref)) * dis[:, None] + b1_ref[...]
  h = jnp.maximum(agg, 0.0)
  n_pad = h.shape[0]
  valid = lax.broadcasted_iota(jnp.int32, (n_pad, 1), 0) < n
  h = jnp.where(valid, h, 0.0)
  t2_ref[...] = _split_cols(h * dis[:, None])


def _tc3_body(n, deg_p_ref, seg_ref, t2_ref, w2_ref, b2_ref, out_ref):
  dis = _dis_from_parts(deg_p_ref[...])[:n]
  agg = (_cat_cols(seg_ref) + _cat_cols(t2_ref))[:n] * dis[:, None]
  out_ref[...] = jnp.dot(
      agg, w2_ref[...], preferred_element_type=jnp.float32) + b2_ref[...]


def kernel(x, edge_index, W1, b1, W2, b2):
  n, d_in = x.shape
  d_hid = W1.shape[1]
  d_out = W2.shape[1]
  e = edge_index.shape[1]

  n_pad = ((n + NS * CH) // (NS * CH)) * NS * CH
  epc = NS * CH
  k_tile = (e + epc - 1) // epc
  k_tile = ((k_tile + 7) // 8) * 8
  e_pad = k_tile * epc

  k_rows = NS * k_tile
  assert e % CH == 0
  e_chunks = e // CH
  ei3 = edge_index.reshape(2, e_chunks, CH)

  kd = ((e_chunks + NC * NS - 1) // (NC * NS) + 7) // 8 * 8
  deg_p = _deg_sc(n_pad, e_chunks, kd)(ei3)

  edges, h1 = pl.pallas_call(
      functools.partial(_tc_a_body, n, k_rows),
      out_shape=(jax.ShapeDtypeStruct((2, k_rows, CH), jnp.int32),
                 jax.ShapeDtypeStruct((n, d_hid), jnp.float32)),
  )(ei3, x, W1)

  hs1, dis_exp = pl.pallas_call(
      functools.partial(_tc1_body, n),
      out_shape=(jax.ShapeDtypeStruct((NC, n_pad, d_hid // 2), jnp.float32),
                 jax.ShapeDtypeStruct((n_pad, d_hid // 2), jnp.float32)),
  )(deg_p, h1)

  t2, seg2 = _gcn_sc(n_pad, d_hid // 2, k_tile, 4, n)(hs1, edges, dis_exp, b1)

  return pl.pallas_call(
      functools.partial(_tc3_body, n),
      out_shape=jax.ShapeDtypeStruct((n, d_out), jnp.float32),
  )(deg_p, seg2, t2, W2, b2)


# speedup vs baseline: 1.5427x; 1.0837x over previous
import functools

import jax
import jax.numpy as jnp
from jax import lax
from jax.experimental import pallas as pl
from jax.experimental.pallas import tpu as pltpu
from jax.experimental.pallas import tpu_sc as plsc

NC = 2
NS = 16
CH = 128


def _segsum_sc(n_pad, d2, k_tile, grp):
  rows_per_tile = n_pad // NS
  rb_chunks = rows_per_tile // CH
  n_groups = k_tile // grp
  mesh = plsc.VectorSubcoreMesh(core_axis_name="c", subcore_axis_name="s")

  @functools.partial(
      pl.kernel,
      out_type=jax.ShapeDtypeStruct((NC, n_pad, d2), jnp.float32),
      mesh=mesh,
      scratch_types=[
          pltpu.VMEM((k_tile, CH), jnp.int32),
          pltpu.VMEM((k_tile, CH), jnp.int32),
          [pltpu.VMEM((CH, d2), jnp.float32) for _ in range(grp)],
          pltpu.VMEM_SHARED((n_pad, d2), jnp.float32),
          pltpu.SemaphoreType.DMA,
          pltpu.SemaphoreType.DMA,
      ],
      compiler_params=pltpu.CompilerParams(use_tc_tiling_on_sc=False),
  )
  def k(table_hbm, edges_hbm, out_hbm,
        src_v, dst_v, rows, acc, gsem, ssem):
    c = lax.axis_index("c")
    s = lax.axis_index("s")
    row0 = s * rows_per_tile
    zbuf = rows[0]

    def zrow(r, carry):
      for i in range(d2 // 16):
        zbuf[r, pl.ds(i * 16, 16)] = jnp.zeros((16,), jnp.float32)
      return carry

    i0 = pltpu.async_copy(edges_hbm.at[0, pl.ds(s * k_tile, k_tile)], src_v,
                          gsem)
    i1 = pltpu.async_copy(edges_hbm.at[1, pl.ds(s * k_tile, k_tile)], dst_v,
                          ssem)
    lax.fori_loop(0, CH, zrow, 0)
    for t in range(rb_chunks):
      pltpu.sync_copy(zbuf, acc.at[pl.ds(row0 + t * CH, CH)])
    i0.wait()
    i1.wait()
    plsc.subcore_barrier()

    def outer(u, carry):
      gds, sds = [], []
      for b in range(grp):
        gds.append(
            pltpu.async_copy(table_hbm.at[c].at[src_v.at[u * grp + b]],
                             rows[b], gsem))
      for b in range(grp):
        gds[b].wait()
        sds.append(
            pltpu.async_copy(rows[b], acc.at[dst_v.at[u * grp + b]], ssem,
                             add=True))
      for b in range(grp):
        sds[b].wait()
      return carry

    lax.fori_loop(0, n_groups, outer, 0)
    plsc.subcore_barrier()

    pltpu.sync_copy(acc.at[pl.ds(row0, rows_per_tile)],
                    out_hbm.at[c, pl.ds(row0, rows_per_tile)])

  return k


def _gcn_sc(n_pad, d2, k_tile, grp, n):
  rows_per_tile = n_pad // NS
  rb_chunks = rows_per_tile // CH
  n_groups = k_tile // grp
  mesh = plsc.VectorSubcoreMesh(core_axis_name="c", subcore_axis_name="s")

  @functools.partial(
      pl.kernel,
      out_type=(jax.ShapeDtypeStruct((NC, n_pad, d2), jnp.float32),
                jax.ShapeDtypeStruct((NC, n_pad, d2), jnp.float32)),
      mesh=mesh,
      scratch_types=[
          pltpu.VMEM((k_tile, CH), jnp.int32),
          pltpu.VMEM((k_tile, CH), jnp.int32),
          [pltpu.VMEM((CH, d2), jnp.float32) for _ in range(grp)],
          pltpu.VMEM((CH, d2), jnp.float32),
          pltpu.VMEM((CH, d2), jnp.float32),
          pltpu.VMEM((CH, d2), jnp.float32),
          pltpu.VMEM((d2,), jnp.float32),
          pltpu.VMEM_SHARED((n_pad, d2), jnp.float32),
          pltpu.SemaphoreType.DMA,
          pltpu.SemaphoreType.DMA,
      ],
      compiler_params=pltpu.CompilerParams(use_tc_tiling_on_sc=False),
  )
  def k(hs1_hbm, edges_hbm, dis_hbm, b1_hbm, t2_hbm, seg2_hbm,
        src_v, dst_v, rows, wbuf, hbuf, dbuf, b1v, acc, gsem, ssem):
    c = lax.axis_index("c")
    s = lax.axis_index("s")
    row0 = s * rows_per_tile
    rsl = pl.ds(row0, rows_per_tile)
    zbuf = rows[0]

    def zrow(r, carry):
      for i in range(d2 // 16):
        zbuf[r, pl.ds(i * 16, 16)] = jnp.zeros((16,), jnp.float32)
      return carry

    def zero_acc():
      for t in range(rb_chunks):
        pltpu.sync_copy(zbuf, acc.at[pl.ds(row0 + t * CH, CH)])

    def segsum(table_hbm, sem):
      def outer(u, carry):
        gds = []
        for b in range(grp):
          gds.append(
              pltpu.async_copy(table_hbm.at[c].at[src_v.at[u * grp + b]],
                               rows[b], sem))
        for b in range(grp):
          gds[b].wait()
          pltpu.sync_copy(rows[b], acc.at[dst_v.at[u * grp + b]], add=True)
        return carry

      lax.fori_loop(0, n_groups, outer, 0)

    i0 = pltpu.async_copy(edges_hbm.at[0, pl.ds(s * k_tile, k_tile)], src_v,
                          gsem)
    i1 = pltpu.async_copy(edges_hbm.at[1, pl.ds(s * k_tile, k_tile)], dst_v,
                          ssem)
    lax.fori_loop(0, CH, zrow, 0)
    zero_acc()
    pltpu.sync_copy(b1_hbm.at[pl.ds(c * d2, d2)], b1v)
    i0.wait()
    i1.wait()
    plsc.subcore_barrier()

    segsum(hs1_hbm, gsem)
    plsc.subcore_barrier()

    def brow(r, carry):
      for i in range(d2 // 16):
        csl = pl.ds(i * 16, 16)
        d = dbuf[r, csl]
        agg = (wbuf[r, csl] + hbuf[r, csl]) * d + b1v[csl]
        wbuf[r, csl] = jnp.maximum(agg, 0.0) * d
      return carry

    for t in range(rb_chunks):
      csl = pl.ds(row0 + t * CH, CH)
      pltpu.sync_copy(acc.at[csl], wbuf)
      pltpu.sync_copy(hs1_hbm.at[c, csl], hbuf)
      pltpu.sync_copy(dis_hbm.at[csl], dbuf)
      lax.fori_loop(0, CH, brow, 0)
      pltpu.sync_copy(wbuf, t2_hbm.at[c, csl])
    plsc.subcore_barrier()

    lax.fori_loop(0, CH, zrow, 0)
    zero_acc()
    plsc.subcore_barrier()
    segsum(t2_hbm, ssem)
    plsc.subcore_barrier()
    pltpu.sync_copy(acc.at[rsl], seg2_hbm.at[c, rsl])

  return k


def _deg_sc(n_pad, e_chunks, kd):
  rows_per_tile = n_pad // NS
  last_cnt = e_chunks - (NC * NS - 1) * kd
  assert 0 < last_cnt <= kd
  mesh = plsc.VectorSubcoreMesh(core_axis_name="c", subcore_axis_name="s")

  @functools.partial(
      pl.kernel,
      out_type=jax.ShapeDtypeStruct((NC, n_pad), jnp.float32),
      mesh=mesh,
      scratch_types=[
          pltpu.VMEM((kd, CH), jnp.int32),
          pltpu.VMEM((CH,), jnp.float32),
          pltpu.VMEM((rows_per_tile,), jnp.float32),
          pltpu.VMEM_SHARED((n_pad,), jnp.float32),
      ],
      compiler_params=pltpu.CompilerParams(use_tc_tiling_on_sc=False),
  )
  def k(edges_hbm, out_hbm, dst_v, ones_v, rb_v, acc):
    c = lax.axis_index("c")
    s = lax.axis_index("s")
    row0 = s * rows_per_tile
    w = c * NS + s
    is_last = w == NC * NS - 1
    n_my = jnp.where(is_last, last_cnt, kd)
    for i in range(CH // 16):
      ones_v[pl.ds(i * 16, 16)] = jnp.ones((16,), jnp.float32)

    def zrow(r, carry):
      rb_v[pl.ds(r * 16, 16)] = jnp.zeros((16,), jnp.float32)
      return carry

    lax.fori_loop(0, rows_per_tile // 16, zrow, 0)
    pltpu.sync_copy(rb_v, acc.at[pl.ds(row0, rows_per_tile)])

    @pl.when(jnp.logical_not(is_last))
    def _():
      pltpu.sync_copy(edges_hbm.at[1, pl.ds(w * kd, kd)], dst_v)

    @pl.when(is_last)
    def _():
      pltpu.sync_copy(edges_hbm.at[1, pl.ds(w * kd, last_cnt)],
                      dst_v.at[pl.ds(0, last_cnt)])

    plsc.subcore_barrier()

    def body(j, carry):
      pltpu.sync_copy(ones_v, acc.at[dst_v.at[j]], add=True)
      return carry

    lax.fori_loop(0, n_my, body, 0)
    plsc.subcore_barrier()
    sl = pl.ds(row0, rows_per_tile)
    pltpu.sync_copy(acc.at[sl], rb_v)
    pltpu.sync_copy(rb_v, out_hbm.at[c, sl])

  return k


def _dis_from_parts(deg_p):
  deg = deg_p[0] + deg_p[1] + 1.0
  return lax.rsqrt(deg)


def _tc_a_body(n, k_rows, ei_ref, x_ref, w1_ref, edges_ref, h1_ref):
  ei = ei_ref[...]
  pad_rows = k_rows - ei.shape[1]
  edges_ref[...] = jnp.concatenate(
      [ei, jnp.full((2, pad_rows, CH), n, jnp.int32)], axis=1)
  h1_ref[...] = jnp.dot(x_ref[...], w1_ref[...],
                        preferred_element_type=jnp.float32)


def _split_cols(x):
  d2 = x.shape[1] // 2
  return jnp.stack([x[:, :d2], x[:, d2:]])


def _cat_cols(ref):
  return jnp.concatenate([ref[0], ref[1]], axis=1)


def _tc1_body(n, deg_p_ref, h1_ref, hs1_ref, dis_ref):
  n_pad = deg_p_ref.shape[1]
  d2 = dis_ref.shape[1]
  dis = _dis_from_parts(deg_p_ref[...])[:n]
  hs = h1_ref[...] * dis[:, None]
  hs = jnp.concatenate(
      [hs, jnp.zeros((n_pad - n, hs.shape[1]), jnp.float32)], axis=0)
  hs1_ref[...] = _split_cols(hs)
  dis_ref[...] = jnp.concatenate(
      [jnp.broadcast_to(dis[:, None], (n, d2)),
       jnp.zeros((n_pad - n, d2), jnp.float32)], axis=0)


def _tc2_body(n, deg_p_ref, seg_ref, hs1_ref, b1_ref, t2_ref):
  dis = _dis_from_parts(deg_p_ref[...])
  agg = (_cat_cols(seg_ref) + _cat_cols(hs1_ref)) * dis[:, None] + b1_ref[...]
  h = jnp.maximum(agg, 0.0)
  n_pad = h.shape[0]
  valid = lax.broadcasted_iota(jnp.int32, (n_pad, 1), 0) < n
  h = jnp.where(valid, h, 0.0)
  t2_ref[...] = _split_cols(h * dis[:, None])


def _tc3_body(n, deg_p_ref, seg_ref, t2_ref, w2_ref, b2_ref, out_ref):
  dis = _dis_from_parts(deg_p_ref[...])[:n]
  agg = (_cat_cols(seg_ref) + _cat_cols(t2_ref))[:n] * dis[:, None]
  out_ref[...] = jnp.dot(
      agg, w2_ref[...], preferred_element_type=jnp.float32) + b2_ref[...]


def kernel(x, edge_index, W1, b1, W2, b2):
  n, d_in = x.shape
  d_hid = W1.shape[1]
  d_out = W2.shape[1]
  e = edge_index.shape[1]

  n_pad = ((n + NS * CH) // (NS * CH)) * NS * CH
  epc = NS * CH
  k_tile = (e + epc - 1) // epc
  k_tile = ((k_tile + 7) // 8) * 8
  e_pad = k_tile * epc

  k_rows = NS * k_tile
  assert e % CH == 0
  e_chunks = e // CH
  ei3 = edge_index.reshape(2, e_chunks, CH)

  kd = ((e_chunks + NC * NS - 1) // (NC * NS) + 7) // 8 * 8
  deg_p = _deg_sc(n_pad, e_chunks, kd)(ei3)

  edges, h1 = pl.pallas_call(
      functools.partial(_tc_a_body, n, k_rows),
      out_shape=(jax.ShapeDtypeStruct((2, k_rows, CH), jnp.int32),
                 jax.ShapeDtypeStruct((n, d_hid), jnp.float32)),
  )(ei3, x, W1)

  hs1, dis_exp = pl.pallas_call(
      functools.partial(_tc1_body, n),
      out_shape=(jax.ShapeDtypeStruct((NC, n_pad, d_hid // 2), jnp.float32),
                 jax.ShapeDtypeStruct((n_pad, d_hid // 2), jnp.float32)),
  )(deg_p, h1)

  t2, seg2 = _gcn_sc(n_pad, d_hid // 2, k_tile, 8, n)(hs1, edges, dis_exp, b1)

  return pl.pallas_call(
      functools.partial(_tc3_body, n),
      out_shape=jax.ShapeDtypeStruct((n, d_out), jnp.float32),
  )(deg_p, seg2, t2, W2, b2)


# speedup vs baseline: 1.5430x; 1.0002x over previous
import functools

import jax
import jax.numpy as jnp
from jax import lax
from jax.experimental import pallas as pl
from jax.experimental.pallas import tpu as pltpu
from jax.experimental.pallas import tpu_sc as plsc

NC = 2
NS = 16
CH = 128


def _gcn_sc(n_pad, d2, k_tile, grp, n):
  rows_per_tile = n_pad // NS
  rb_chunks = rows_per_tile // CH
  n_groups = k_tile // grp
  mesh = plsc.VectorSubcoreMesh(core_axis_name="c", subcore_axis_name="s")

  @functools.partial(
      pl.kernel,
      out_type=(jax.ShapeDtypeStruct((NC, n_pad, d2), jnp.float32),
                jax.ShapeDtypeStruct((NC, n_pad, d2), jnp.float32)),
      mesh=mesh,
      scratch_types=[
          pltpu.VMEM((k_tile, CH), jnp.int32),
          pltpu.VMEM((k_tile, CH), jnp.int32),
          [pltpu.VMEM((CH, d2), jnp.float32) for _ in range(grp)],
          pltpu.VMEM((CH, d2), jnp.float32),
          pltpu.VMEM((CH, d2), jnp.float32),
          pltpu.VMEM((CH, d2), jnp.float32),
          pltpu.VMEM((d2,), jnp.float32),
          pltpu.VMEM_SHARED((n_pad, d2), jnp.float32),
          pltpu.SemaphoreType.DMA,
          pltpu.SemaphoreType.DMA,
      ],
      compiler_params=pltpu.CompilerParams(use_tc_tiling_on_sc=False),
  )
  def k(hs1_hbm, edges_hbm, dis_hbm, b1_hbm, t2_hbm, seg2_hbm,
        src_v, dst_v, rows, wbuf, hbuf, dbuf, b1v, acc, gsem, ssem):
    c = lax.axis_index("c")
    s = lax.axis_index("s")
    row0 = s * rows_per_tile
    rsl = pl.ds(row0, rows_per_tile)
    zbuf = rows[0]

    def zrow(r, carry):
      for i in range(d2 // 16):
        zbuf[r, pl.ds(i * 16, 16)] = jnp.zeros((16,), jnp.float32)
      return carry

    def zero_acc():
      for t in range(rb_chunks):
        pltpu.sync_copy(zbuf, acc.at[pl.ds(row0 + t * CH, CH)])

    def segsum(table_hbm, sem):
      def outer(u, carry):
        gds = []
        for b in range(grp):
          gds.append(
              pltpu.async_copy(table_hbm.at[c].at[src_v.at[u * grp + b]],
                               rows[b], sem))
        for b in range(grp):
          gds[b].wait()
          pltpu.sync_copy(rows[b], acc.at[dst_v.at[u * grp + b]], add=True)
        return carry

      lax.fori_loop(0, n_groups, outer, 0)

    i0 = pltpu.async_copy(edges_hbm.at[0, pl.ds(s * k_tile, k_tile)], src_v,
                          gsem)
    i1 = pltpu.async_copy(edges_hbm.at[1, pl.ds(s * k_tile, k_tile)], dst_v,
                          ssem)
    lax.fori_loop(0, CH, zrow, 0)
    zero_acc()
    pltpu.sync_copy(b1_hbm.at[pl.ds(c * d2, d2)], b1v)
    i0.wait()
    i1.wait()
    plsc.subcore_barrier()

    segsum(hs1_hbm, gsem)
    plsc.subcore_barrier()

    def brow(r, carry):
      for i in range(d2 // 16):
        csl = pl.ds(i * 16, 16)
        d = dbuf[r, csl]
        agg = (wbuf[r, csl] + hbuf[r, csl]) * d + b1v[csl]
        wbuf[r, csl] = jnp.maximum(agg, 0.0) * d
      return carry

    for t in range(rb_chunks):
      csl = pl.ds(row0 + t * CH, CH)
      pltpu.sync_copy(acc.at[csl], wbuf)
      pltpu.sync_copy(hs1_hbm.at[c, csl], hbuf)
      pltpu.sync_copy(dis_hbm.at[csl], dbuf)
      lax.fori_loop(0, CH, brow, 0)
      pltpu.sync_copy(wbuf, t2_hbm.at[c, csl])
    plsc.subcore_barrier()

    lax.fori_loop(0, CH, zrow, 0)
    zero_acc()
    plsc.subcore_barrier()
    segsum(t2_hbm, ssem)
    plsc.subcore_barrier()
    pltpu.sync_copy(acc.at[rsl], seg2_hbm.at[c, rsl])

  return k


def _deg_sc(n_pad, e_chunks, kd):
  rows_per_tile = n_pad // NS
  last_cnt = e_chunks - (NC * NS - 1) * kd
  assert 0 < last_cnt <= kd
  mesh = plsc.VectorSubcoreMesh(core_axis_name="c", subcore_axis_name="s")

  @functools.partial(
      pl.kernel,
      out_type=jax.ShapeDtypeStruct((NC, n_pad), jnp.float32),
      mesh=mesh,
      scratch_types=[
          pltpu.VMEM((kd, CH), jnp.int32),
          pltpu.VMEM((CH,), jnp.float32),
          pltpu.VMEM((rows_per_tile,), jnp.float32),
          pltpu.VMEM_SHARED((n_pad,), jnp.float32),
      ],
      compiler_params=pltpu.CompilerParams(use_tc_tiling_on_sc=False),
  )
  def k(edges_hbm, out_hbm, dst_v, ones_v, rb_v, acc):
    c = lax.axis_index("c")
    s = lax.axis_index("s")
    row0 = s * rows_per_tile
    w = c * NS + s
    is_last = w == NC * NS - 1
    n_my = jnp.where(is_last, last_cnt, kd)
    for i in range(CH // 16):
      ones_v[pl.ds(i * 16, 16)] = jnp.ones((16,), jnp.float32)

    def zrow(r, carry):
      rb_v[pl.ds(r * 16, 16)] = jnp.zeros((16,), jnp.float32)
      return carry

    lax.fori_loop(0, rows_per_tile // 16, zrow, 0)
    pltpu.sync_copy(rb_v, acc.at[pl.ds(row0, rows_per_tile)])

    @pl.when(jnp.logical_not(is_last))
    def _():
      pltpu.sync_copy(edges_hbm.at[1, pl.ds(w * kd, kd)], dst_v)

    @pl.when(is_last)
    def _():
      pltpu.sync_copy(edges_hbm.at[1, pl.ds(w * kd, last_cnt)],
                      dst_v.at[pl.ds(0, last_cnt)])

    plsc.subcore_barrier()

    def body(j, carry):
      pltpu.sync_copy(ones_v, acc.at[dst_v.at[j]], add=True)
      return carry

    lax.fori_loop(0, n_my, body, 0)
    plsc.subcore_barrier()
    sl = pl.ds(row0, rows_per_tile)
    pltpu.sync_copy(acc.at[sl], rb_v)
    pltpu.sync_copy(rb_v, out_hbm.at[c, sl])

  return k


def _dis_from_parts(deg_p):
  deg = deg_p[0] + deg_p[1] + 1.0
  return lax.rsqrt(deg)


def _tc_a_body(n, k_rows, ei_ref, x_ref, w1_ref, edges_ref, h1_ref):
  ei = ei_ref[...]
  pad_rows = k_rows - ei.shape[1]
  edges_ref[...] = jnp.concatenate(
      [ei, jnp.full((2, pad_rows, CH), n, jnp.int32)], axis=1)
  h1_ref[...] = jnp.dot(x_ref[...], w1_ref[...],
                        preferred_element_type=jnp.float32)


def _split_cols(x):
  d2 = x.shape[1] // 2
  return jnp.stack([x[:, :d2], x[:, d2:]])


def _cat_cols(ref):
  return jnp.concatenate([ref[0], ref[1]], axis=1)


def _tc1_body(n, deg_p_ref, h1_ref, hs1_ref, dis_ref):
  n_pad = deg_p_ref.shape[1]
  d2 = dis_ref.shape[1]
  dis = _dis_from_parts(deg_p_ref[...])[:n]
  hs = h1_ref[...] * dis[:, None]
  hs = jnp.concatenate(
      [hs, jnp.zeros((n_pad - n, hs.shape[1]), jnp.float32)], axis=0)
  hs1_ref[...] = _split_cols(hs)
  dis_ref[...] = jnp.concatenate(
      [jnp.broadcast_to(dis[:, None], (n, d2)),
       jnp.zeros((n_pad - n, d2), jnp.float32)], axis=0)


def _tc3_body(n, deg_p_ref, seg_ref, t2_ref, w2_ref, b2_ref, out_ref):
  dis = _dis_from_parts(deg_p_ref[...])[:n]
  agg = (_cat_cols(seg_ref) + _cat_cols(t2_ref))[:n] * dis[:, None]
  out_ref[...] = jnp.dot(
      agg, w2_ref[...], preferred_element_type=jnp.float32) + b2_ref[...]


def kernel(x, edge_index, W1, b1, W2, b2):
  n, d_in = x.shape
  d_hid = W1.shape[1]
  d_out = W2.shape[1]
  e = edge_index.shape[1]

  n_pad = ((n + NS * CH) // (NS * CH)) * NS * CH
  epc = NS * CH
  k_tile = (e + epc - 1) // epc
  k_tile = ((k_tile + 7) // 8) * 8
  e_pad = k_tile * epc

  k_rows = NS * k_tile
  assert e % CH == 0
  e_chunks = e // CH
  ei3 = edge_index.reshape(2, e_chunks, CH)

  kd = ((e_chunks + NC * NS - 1) // (NC * NS) + 7) // 8 * 8
  deg_p = _deg_sc(n_pad, e_chunks, kd)(ei3)

  edges, h1 = pl.pallas_call(
      functools.partial(_tc_a_body, n, k_rows),
      out_shape=(jax.ShapeDtypeStruct((2, k_rows, CH), jnp.int32),
                 jax.ShapeDtypeStruct((n, d_hid), jnp.float32)),
  )(ei3, x, W1)

  hs1, dis_exp = pl.pallas_call(
      functools.partial(_tc1_body, n),
      out_shape=(jax.ShapeDtypeStruct((NC, n_pad, d_hid // 2), jnp.float32),
                 jax.ShapeDtypeStruct((n_pad, d_hid // 2), jnp.float32)),
  )(deg_p, h1)

  t2, seg2 = _gcn_sc(n_pad, d_hid // 2, k_tile, 8, n)(hs1, edges, dis_exp, b1)

  return pl.pallas_call(
      functools.partial(_tc3_body, n),
      out_shape=jax.ShapeDtypeStruct((n, d_out), jnp.float32),
  )(deg_p, seg2, t2, W2, b2)
